# baseline (speedup 1.0000x reference)
import jax
import jax.numpy as jnp
from jax.experimental import pallas as pl
from jax.experimental.pallas import tpu as pltpu

NE = 8
TOPK = 2
H = 1024
INTER = 4096
KI_BLK = 1024
KI = INTER // KI_BLK
T = 256
N = 2048
R = N * TOPK + NE * T
RT = R // T
RT_PAD = 32
GR = 1024


def _gelu(x):
    inner = 0.7978845608028654 * (x + 0.044715 * (x * x * x))
    return 0.5 * x * (1.0 + jnp.tanh(inner))


def _router_kernel(x_ref, wr_ref, w1_ref, w2_ref, pos1_ref, pos2_ref,
                   te_ref, tot_ref, aux_ref):
    n = x_ref.shape[0]
    logits = jax.lax.dot_general(
        x_ref[...], wr_ref[...], (((1,), (1,)), ((), ())),
        preferred_element_type=jnp.float32)
    m = jnp.max(logits, axis=-1, keepdims=True)
    ex = jnp.exp(logits - m)
    probs = ex / jnp.sum(ex, axis=-1, keepdims=True)

    lane = jax.lax.broadcasted_iota(jnp.int32, (n, NE), 1)
    m1 = jnp.max(probs, axis=-1, keepdims=True)
    a1 = jnp.min(jnp.where(probs >= m1, lane, NE), axis=-1, keepdims=True)
    oh1 = (lane == a1).astype(jnp.float32)
    p2 = jnp.where(lane == a1, -1.0, probs)
    m2 = jnp.max(p2, axis=-1, keepdims=True)
    a2 = jnp.min(jnp.where(p2 >= m2, lane, NE), axis=-1, keepdims=True)
    oh2 = (lane == a2).astype(jnp.float32)

    s = m1 + m2
    w1_ref[...] = m1 / s
    w2_ref[...] = m2 / s

    c = oh1 + oh2
    ri = jax.lax.broadcasted_iota(jnp.int32, (n, n), 0)
    ci = jax.lax.broadcasted_iota(jnp.int32, (n, n), 1)
    tri = (ci < ri).astype(jnp.float32)
    rank = jax.lax.dot_general(
        tri, c, (((1,), (0,)), ((), ())),
        preferred_element_type=jnp.float32)
    counts = jnp.sum(c, axis=0, keepdims=True)
    pc = jnp.ceil(counts / T) * T
    er = jax.lax.broadcasted_iota(jnp.int32, (NE, NE), 0)
    ec = jax.lax.broadcasted_iota(jnp.int32, (NE, NE), 1)
    off = jax.lax.dot_general(
        pc, (er < ec).astype(jnp.float32), (((1,), (0,)), ((), ())),
        preferred_element_type=jnp.float32)
    ends = jax.lax.dot_general(
        pc, (er <= ec).astype(jnp.float32), (((1,), (0,)), ((), ())),
        preferred_element_type=jnp.float32)

    base = off + rank
    pos1_ref[...] = jnp.sum(base * oh1, axis=1, keepdims=True).astype(jnp.int32)
    pos2_ref[...] = jnp.sum(base * oh2, axis=1, keepdims=True).astype(jnp.int32)

    rt = (jax.lax.broadcasted_iota(jnp.int32, (RT_PAD, NE), 0) * T).astype(
        jnp.float32)
    te = jnp.sum((rt >= ends).astype(jnp.int32), axis=1, keepdims=True)
    te_ref[...] = jnp.minimum(te, NE - 1)
    tot_ref[...] = ends[:, NE - 1:NE].astype(jnp.int32)

    f = counts / (n * TOPK)
    pmean = jnp.mean(probs, axis=0, keepdims=True)
    aux_ref[...] = jnp.full((1, 1), NE * jnp.sum(f * pmean), jnp.float32)


def _gather_kernel(pos1_ref, pos2_ref, x_ref, xs_ref, xb_ref):
    j = pl.program_id(0)

    @pl.when(j == 0)
    def _():
        xb_ref[...] = x_ref[...].astype(jnp.bfloat16)

    ji = jax.lax.broadcasted_iota(jnp.int32, (GR, N), 0) + j * GR
    P = ((ji == pos1_ref[...]) | (ji == pos2_ref[...])).astype(jnp.bfloat16)
    xs_ref[...] = jax.lax.dot_general(
        P, xb_ref[...], (((1,), (0,)), ((), ())),
        preferred_element_type=jnp.float32).astype(jnp.bfloat16)


def _up_kernel(te_ref, tot_ref, xs_ref, wup_ref,
               pos1_ref, pos2_ref, w1_ref, w2_ref, h_ref, wub_ref):
    r = pl.program_id(0)

    changed = (r == 0) | (te_ref[r] != te_ref[jnp.maximum(r - 1, 0)])

    @pl.when(changed)
    def _():
        wub_ref[...] = wup_ref[0].astype(jnp.bfloat16)

    @pl.when(r * T < tot_ref[0])
    def _():
        h = jax.lax.dot_general(
            xs_ref[...], wub_ref[...], (((1,), (1,)), ((), ())),
            preferred_element_type=jnp.float32)
        ji = jax.lax.broadcasted_iota(jnp.int32, (T, N), 0) + r * T
        wrow = jnp.sum(
            jnp.where(ji == pos1_ref[...], w1_ref[...], 0.0)
            + jnp.where(ji == pos2_ref[...], w2_ref[...], 0.0),
            axis=1, keepdims=True)
        h_ref[...] = (_gelu(h) * wrow).astype(jnp.bfloat16)


def _down_kernel(te_ref, tot_ref, h_ref, wdn_ref, y_ref, wdb_ref):
    r = pl.program_id(0)
    changed = (r == 0) | (te_ref[r] != te_ref[jnp.maximum(r - 1, 0)])

    @pl.when(changed)
    def _():
        wdb_ref[...] = wdn_ref[0].astype(jnp.bfloat16)

    @pl.when(r * T < tot_ref[0])
    def _():
        part = jax.lax.dot_general(
            h_ref[...], wdb_ref[...], (((1,), (1,)), ((), ())),
            preferred_element_type=jnp.float32)
        y_ref[...] = part.astype(jnp.bfloat16)


def _combine_kernel(pos1_ref, pos2_ref, y_ref, out_ref):
    j = pl.program_id(0)
    ji = jax.lax.broadcasted_iota(jnp.int32, (N, GR), 1) + j * GR
    C = ((ji == pos1_ref[...]) | (ji == pos2_ref[...])).astype(jnp.bfloat16)
    part = jax.lax.dot_general(
        C, y_ref[...], (((1,), (0,)), ((), ())),
        preferred_element_type=jnp.float32)

    @pl.when(j == 0)
    def _():
        out_ref[...] = part

    @pl.when(j > 0)
    def _():
        out_ref[...] += part


def kernel(hidden_states, W_router, W_up, W_down):
    batch, seq, hidden = hidden_states.shape
    n = batch * seq
    flat = hidden_states.reshape(n, hidden)

    w1c, w2c, pos1c, pos2c, te, tot, aux = pl.pallas_call(
        _router_kernel,
        out_shape=[
            jax.ShapeDtypeStruct((n, 1), jnp.float32),
            jax.ShapeDtypeStruct((n, 1), jnp.float32),
            jax.ShapeDtypeStruct((n, 1), jnp.int32),
            jax.ShapeDtypeStruct((n, 1), jnp.int32),
            jax.ShapeDtypeStruct((RT_PAD, 1), jnp.int32),
            jax.ShapeDtypeStruct((1, 1), jnp.int32),
            jax.ShapeDtypeStruct((1, 1), jnp.float32),
        ],
    )(flat, W_router)

    pos1r = pos1c.reshape(1, n)
    pos2r = pos2c.reshape(1, n)
    w1r = w1c.reshape(1, n)
    w2r = w2c.reshape(1, n)

    xs = pl.pallas_call(
        _gather_kernel,
        grid=(R // GR,),
        in_specs=[
            pl.BlockSpec((1, n), lambda j: (0, 0)),
            pl.BlockSpec((1, n), lambda j: (0, 0)),
            pl.BlockSpec((n, H), lambda j: (0, 0)),
        ],
        out_specs=pl.BlockSpec((GR, H), lambda j: (j, 0)),
        out_shape=jax.ShapeDtypeStruct((R, H), jnp.bfloat16),
        scratch_shapes=[pltpu.VMEM((n, H), jnp.bfloat16)],
    )(pos1r, pos2r, flat)

    te_flat = te.reshape(RT_PAD)
    tot_flat = tot.reshape(1)

    h = pl.pallas_call(
        _up_kernel,
        grid_spec=pltpu.PrefetchScalarGridSpec(
            num_scalar_prefetch=2,
            grid=(RT,),
            in_specs=[
                pl.BlockSpec((T, H), lambda r, te, tot: (r, 0)),
                pl.BlockSpec((1, INTER, H), lambda r, te, tot: (te[r], 0, 0)),
                pl.BlockSpec((1, n), lambda r, te, tot: (0, 0)),
                pl.BlockSpec((1, n), lambda r, te, tot: (0, 0)),
                pl.BlockSpec((1, n), lambda r, te, tot: (0, 0)),
                pl.BlockSpec((1, n), lambda r, te, tot: (0, 0)),
            ],
            out_specs=pl.BlockSpec((T, INTER), lambda r, te, tot: (r, 0)),
            scratch_shapes=[pltpu.VMEM((INTER, H), jnp.bfloat16)],
        ),
        out_shape=jax.ShapeDtypeStruct((R, INTER), jnp.bfloat16),
    )(te_flat, tot_flat, xs, W_up, pos1r, pos2r, w1r, w2r)

    y = pl.pallas_call(
        _down_kernel,
        grid_spec=pltpu.PrefetchScalarGridSpec(
            num_scalar_prefetch=2,
            grid=(RT,),
            in_specs=[
                pl.BlockSpec((T, INTER), lambda r, te, tot: (r, 0)),
                pl.BlockSpec((1, H, INTER), lambda r, te, tot: (te[r], 0, 0)),
            ],
            out_specs=pl.BlockSpec((T, H), lambda r, te, tot: (r, 0)),
            scratch_shapes=[pltpu.VMEM((H, INTER), jnp.bfloat16)],
        ),
        out_shape=jax.ShapeDtypeStruct((R, H), jnp.bfloat16),
    )(te_flat, tot_flat, h, W_down)

    out = pl.pallas_call(
        _combine_kernel,
        grid=(R // GR,),
        in_specs=[
            pl.BlockSpec((n, 1), lambda j: (0, 0)),
            pl.BlockSpec((n, 1), lambda j: (0, 0)),
            pl.BlockSpec((GR, H), lambda j: (j, 0)),
        ],
        out_specs=pl.BlockSpec((n, H), lambda j: (0, 0)),
        out_shape=jax.ShapeDtypeStruct((n, H), jnp.float32),
    )(pos1c, pos2c, y)

    return out.reshape(batch, seq, hidden), aux[0, 0]


# speedup vs baseline: 1.1314x; 1.1314x over previous
import jax
import jax.numpy as jnp
from jax.experimental import pallas as pl
from jax.experimental.pallas import tpu as pltpu

NE = 8
TOPK = 2
H = 1024
INTER = 4096
KI_BLK = 1024
KI = INTER // KI_BLK
T = 256
N = 2048
R = N * TOPK + NE * T
RT = R // T
RT_PAD = 32
GR = 1024


def _gelu(x):
    inner = 0.7978845608028654 * (x + 0.044715 * (x * x * x))
    return 0.5 * x * (1.0 + jnp.tanh(inner))


def _router_kernel(x_ref, wr_ref, w1_ref, w2_ref, pos1_ref, pos2_ref,
                   te_ref, tot_ref, aux_ref):
    n = x_ref.shape[0]
    logits = jax.lax.dot_general(
        x_ref[...], wr_ref[...], (((1,), (1,)), ((), ())),
        preferred_element_type=jnp.float32)
    m = jnp.max(logits, axis=-1, keepdims=True)
    ex = jnp.exp(logits - m)
    probs = ex / jnp.sum(ex, axis=-1, keepdims=True)

    lane = jax.lax.broadcasted_iota(jnp.int32, (n, NE), 1)
    m1 = jnp.max(probs, axis=-1, keepdims=True)
    a1 = jnp.min(jnp.where(probs >= m1, lane, NE), axis=-1, keepdims=True)
    oh1 = (lane == a1).astype(jnp.float32)
    p2 = jnp.where(lane == a1, -1.0, probs)
    m2 = jnp.max(p2, axis=-1, keepdims=True)
    a2 = jnp.min(jnp.where(p2 >= m2, lane, NE), axis=-1, keepdims=True)
    oh2 = (lane == a2).astype(jnp.float32)

    s = m1 + m2
    w1_ref[...] = m1 / s
    w2_ref[...] = m2 / s

    c = oh1 + oh2
    ri = jax.lax.broadcasted_iota(jnp.int32, (n, n), 0)
    ci = jax.lax.broadcasted_iota(jnp.int32, (n, n), 1)
    tri = (ci < ri).astype(jnp.float32)
    rank = jax.lax.dot_general(
        tri, c, (((1,), (0,)), ((), ())),
        preferred_element_type=jnp.float32)
    counts = jnp.sum(c, axis=0, keepdims=True)
    pc = jnp.ceil(counts / T) * T
    er = jax.lax.broadcasted_iota(jnp.int32, (NE, NE), 0)
    ec = jax.lax.broadcasted_iota(jnp.int32, (NE, NE), 1)
    off = jax.lax.dot_general(
        pc, (er < ec).astype(jnp.float32), (((1,), (0,)), ((), ())),
        preferred_element_type=jnp.float32)
    ends = jax.lax.dot_general(
        pc, (er <= ec).astype(jnp.float32), (((1,), (0,)), ((), ())),
        preferred_element_type=jnp.float32)

    base = off + rank
    pos1_ref[...] = jnp.sum(base * oh1, axis=1, keepdims=True).astype(jnp.int32)
    pos2_ref[...] = jnp.sum(base * oh2, axis=1, keepdims=True).astype(jnp.int32)

    rt = (jax.lax.broadcasted_iota(jnp.int32, (RT_PAD, NE), 0) * T).astype(
        jnp.float32)
    te = jnp.sum((rt >= ends).astype(jnp.int32), axis=1, keepdims=True)
    te_ref[...] = jnp.minimum(te, NE - 1)
    tot_ref[...] = ends[:, NE - 1:NE].astype(jnp.int32)

    f = counts / (n * TOPK)
    pmean = jnp.mean(probs, axis=0, keepdims=True)
    aux_ref[...] = jnp.full((1, 1), NE * jnp.sum(f * pmean), jnp.float32)


def _gather_kernel(pos1_ref, pos2_ref, x_ref, xs_ref):
    j = pl.program_id(0)
    ji = jax.lax.broadcasted_iota(jnp.int32, (GR, N), 0) + j * GR
    P = ((ji == pos1_ref[...]) | (ji == pos2_ref[...])).astype(jnp.bfloat16)
    xs_ref[...] = jax.lax.dot_general(
        P, x_ref[...].astype(jnp.bfloat16), (((1,), (0,)), ((), ())),
        preferred_element_type=jnp.float32).astype(jnp.bfloat16)


def _up_kernel(te_ref, tot_ref, xs_ref, wup_ref,
               pos1_ref, pos2_ref, w1_ref, w2_ref, h_ref):
    r = pl.program_id(0)

    @pl.when(r * T < tot_ref[0])
    def _():
        wup = wup_ref[0].astype(jnp.bfloat16)
        h = jax.lax.dot_general(
            xs_ref[...], wup, (((1,), (1,)), ((), ())),
            preferred_element_type=jnp.float32)
        ji = jax.lax.broadcasted_iota(jnp.int32, (T, N), 0) + r * T
        wrow = jnp.sum(
            jnp.where(ji == pos1_ref[...], w1_ref[...], 0.0)
            + jnp.where(ji == pos2_ref[...], w2_ref[...], 0.0),
            axis=1, keepdims=True)
        h_ref[...] = (_gelu(h) * wrow).astype(jnp.bfloat16)


def _down_combine_kernel(te_ref, tot_ref, h_ref, wdn_ref,
                         pos1_ref, pos2_ref, out_ref):
    r = pl.program_id(0)

    @pl.when(r == 0)
    def _():
        out_ref[...] = jnp.zeros_like(out_ref)

    @pl.when(r * T < tot_ref[0])
    def _():
        wdn = wdn_ref[0].astype(jnp.bfloat16)
        part = jax.lax.dot_general(
            h_ref[...], wdn, (((1,), (1,)), ((), ())),
            preferred_element_type=jnp.float32)
        ji = jax.lax.broadcasted_iota(jnp.int32, (N, T), 1) + r * T
        C = ((ji == pos1_ref[...]) | (ji == pos2_ref[...])).astype(jnp.bfloat16)
        out_ref[...] += jax.lax.dot_general(
            C, part.astype(jnp.bfloat16), (((1,), (0,)), ((), ())),
            preferred_element_type=jnp.float32)


def kernel(hidden_states, W_router, W_up, W_down):
    batch, seq, hidden = hidden_states.shape
    n = batch * seq
    flat = hidden_states.reshape(n, hidden)

    w1c, w2c, pos1c, pos2c, te, tot, aux = pl.pallas_call(
        _router_kernel,
        out_shape=[
            jax.ShapeDtypeStruct((n, 1), jnp.float32),
            jax.ShapeDtypeStruct((n, 1), jnp.float32),
            jax.ShapeDtypeStruct((n, 1), jnp.int32),
            jax.ShapeDtypeStruct((n, 1), jnp.int32),
            jax.ShapeDtypeStruct((RT_PAD, 1), jnp.int32),
            jax.ShapeDtypeStruct((1, 1), jnp.int32),
            jax.ShapeDtypeStruct((1, 1), jnp.float32),
        ],
    )(flat, W_router)

    pos1r = pos1c.reshape(1, n)
    pos2r = pos2c.reshape(1, n)
    w1r = w1c.reshape(1, n)
    w2r = w2c.reshape(1, n)

    xs = pl.pallas_call(
        _gather_kernel,
        grid=(R // GR,),
        in_specs=[
            pl.BlockSpec((1, n), lambda j: (0, 0)),
            pl.BlockSpec((1, n), lambda j: (0, 0)),
            pl.BlockSpec((n, H), lambda j: (0, 0)),
        ],
        out_specs=pl.BlockSpec((GR, H), lambda j: (j, 0)),
        out_shape=jax.ShapeDtypeStruct((R, H), jnp.bfloat16),
    )(pos1r, pos2r, flat)

    te_flat = te.reshape(RT_PAD)
    tot_flat = tot.reshape(1)

    h = pl.pallas_call(
        _up_kernel,
        grid_spec=pltpu.PrefetchScalarGridSpec(
            num_scalar_prefetch=2,
            grid=(RT,),
            in_specs=[
                pl.BlockSpec((T, H), lambda r, te, tot: (r, 0)),
                pl.BlockSpec((1, INTER, H), lambda r, te, tot: (te[r], 0, 0)),
                pl.BlockSpec((1, n), lambda r, te, tot: (0, 0)),
                pl.BlockSpec((1, n), lambda r, te, tot: (0, 0)),
                pl.BlockSpec((1, n), lambda r, te, tot: (0, 0)),
                pl.BlockSpec((1, n), lambda r, te, tot: (0, 0)),
            ],
            out_specs=pl.BlockSpec((T, INTER), lambda r, te, tot: (r, 0)),
        ),
        out_shape=jax.ShapeDtypeStruct((R, INTER), jnp.bfloat16),
    )(te_flat, tot_flat, xs, W_up, pos1r, pos2r, w1r, w2r)

    out = pl.pallas_call(
        _down_combine_kernel,
        grid_spec=pltpu.PrefetchScalarGridSpec(
            num_scalar_prefetch=2,
            grid=(RT,),
            in_specs=[
                pl.BlockSpec((T, INTER), lambda r, te, tot: (r, 0)),
                pl.BlockSpec((1, H, INTER), lambda r, te, tot: (te[r], 0, 0)),
                pl.BlockSpec((n, 1), lambda r, te, tot: (0, 0)),
                pl.BlockSpec((n, 1), lambda r, te, tot: (0, 0)),
            ],
            out_specs=pl.BlockSpec((n, H), lambda r, te, tot: (0, 0)),
        ),
        out_shape=jax.ShapeDtypeStruct((n, H), jnp.float32),
    )(te_flat, tot_flat, h, W_down, pos1c, pos2c)

    return out.reshape(batch, seq, hidden), aux[0, 0]


# speedup vs baseline: 1.1559x; 1.0216x over previous
import functools

import jax
import jax.numpy as jnp
from jax import lax
from jax.experimental import pallas as pl
from jax.experimental.pallas import tpu as pltpu
from jax.experimental.pallas import tpu_sc as plsc

NE = 8
TOPK = 2
H = 1024
INTER = 4096
KI_BLK = 1024
KI = INTER // KI_BLK
T = 256
N = 2048
R = N * TOPK + NE * T
RT = R // T
RT_PAD = 32
GR = 1024


def _gelu(x):
    inner = 0.7978845608028654 * (x + 0.044715 * (x * x * x))
    return 0.5 * x * (1.0 + jnp.tanh(inner))


def _router_kernel(x_ref, wr_ref, w1_ref, w2_ref, pos1_ref, pos2_ref,
                   te_ref, tot_ref, aux_ref):
    n = x_ref.shape[0]
    logits = jax.lax.dot_general(
        x_ref[...], wr_ref[...], (((1,), (1,)), ((), ())),
        preferred_element_type=jnp.float32)
    m = jnp.max(logits, axis=-1, keepdims=True)
    ex = jnp.exp(logits - m)
    probs = ex / jnp.sum(ex, axis=-1, keepdims=True)

    lane = jax.lax.broadcasted_iota(jnp.int32, (n, NE), 1)
    m1 = jnp.max(probs, axis=-1, keepdims=True)
    a1 = jnp.min(jnp.where(probs >= m1, lane, NE), axis=-1, keepdims=True)
    oh1 = (lane == a1).astype(jnp.float32)
    p2 = jnp.where(lane == a1, -1.0, probs)
    m2 = jnp.max(p2, axis=-1, keepdims=True)
    a2 = jnp.min(jnp.where(p2 >= m2, lane, NE), axis=-1, keepdims=True)
    oh2 = (lane == a2).astype(jnp.float32)

    s = m1 + m2
    w1_ref[...] = m1 / s
    w2_ref[...] = m2 / s

    c = oh1 + oh2
    ri = jax.lax.broadcasted_iota(jnp.int32, (n, n), 0)
    ci = jax.lax.broadcasted_iota(jnp.int32, (n, n), 1)
    tri = (ci < ri).astype(jnp.float32)
    rank = jax.lax.dot_general(
        tri, c, (((1,), (0,)), ((), ())),
        preferred_element_type=jnp.float32)
    counts = jnp.sum(c, axis=0, keepdims=True)
    pc = jnp.ceil(counts / T) * T
    er = jax.lax.broadcasted_iota(jnp.int32, (NE, NE), 0)
    ec = jax.lax.broadcasted_iota(jnp.int32, (NE, NE), 1)
    off = jax.lax.dot_general(
        pc, (er < ec).astype(jnp.float32), (((1,), (0,)), ((), ())),
        preferred_element_type=jnp.float32)
    ends = jax.lax.dot_general(
        pc, (er <= ec).astype(jnp.float32), (((1,), (0,)), ((), ())),
        preferred_element_type=jnp.float32)

    base = off + rank
    pos1_ref[...] = jnp.sum(base * oh1, axis=1, keepdims=True).astype(jnp.int32)
    pos2_ref[...] = jnp.sum(base * oh2, axis=1, keepdims=True).astype(jnp.int32)

    rt = (jax.lax.broadcasted_iota(jnp.int32, (RT_PAD, NE), 0) * T).astype(
        jnp.float32)
    te = jnp.sum((rt >= ends).astype(jnp.int32), axis=1, keepdims=True)
    te_ref[...] = jnp.minimum(te, NE - 1)
    tot_ref[...] = ends[:, NE - 1:NE].astype(jnp.int32)

    f = counts / (n * TOPK)
    pmean = jnp.mean(probs, axis=0, keepdims=True)
    aux_ref[...] = jnp.full((1, 1), NE * jnp.sum(f * pmean), jnp.float32)


NWORK = 32
TPW = N // NWORK


def _sc_scatter_body(pos1_ref, pos2_ref, x_ref, xs_ref,
                     idx1_v, idx2_v, rows_v, sem):
    wid = lax.axis_index("s") * 2 + lax.axis_index("c")
    base = wid * TPW
    pltpu.sync_copy(pos1_ref.at[pl.ds(base, TPW)], idx1_v)
    pltpu.sync_copy(pos2_ref.at[pl.ds(base, TPW)], idx2_v)
    pltpu.sync_copy(x_ref.at[pl.ds(base, TPW)], rows_v)
    pltpu.async_copy(rows_v, xs_ref.at[idx1_v], sem).wait()
    pltpu.async_copy(rows_v, xs_ref.at[idx2_v], sem).wait()


def _up_kernel(te_ref, tot_ref, xs_ref, wup_ref,
               pos1_ref, pos2_ref, w1_ref, w2_ref, h_ref):
    r = pl.program_id(0)

    @pl.when(r * T < tot_ref[0])
    def _():
        wup = wup_ref[0].astype(jnp.bfloat16)
        h = jax.lax.dot_general(
            xs_ref[...].astype(jnp.bfloat16), wup, (((1,), (1,)), ((), ())),
            preferred_element_type=jnp.float32)
        ji = jax.lax.broadcasted_iota(jnp.int32, (T, N), 0) + r * T
        wrow = jnp.sum(
            jnp.where(ji == pos1_ref[...], w1_ref[...], 0.0)
            + jnp.where(ji == pos2_ref[...], w2_ref[...], 0.0),
            axis=1, keepdims=True)
        h_ref[...] = jnp.where(wrow > 0.0, _gelu(h) * wrow,
                               0.0).astype(jnp.bfloat16)


def _down_combine_kernel(te_ref, tot_ref, h_ref, wdn_ref,
                         pos1_ref, pos2_ref, out_ref):
    r = pl.program_id(0)

    @pl.when(r == 0)
    def _():
        out_ref[...] = jnp.zeros_like(out_ref)

    @pl.when(r * T < tot_ref[0])
    def _():
        wdn = wdn_ref[0].astype(jnp.bfloat16)
        part = jax.lax.dot_general(
            h_ref[...], wdn, (((1,), (1,)), ((), ())),
            preferred_element_type=jnp.float32)
        ji = jax.lax.broadcasted_iota(jnp.int32, (N, T), 1) + r * T
        C = ((ji == pos1_ref[...]) | (ji == pos2_ref[...])).astype(jnp.bfloat16)
        out_ref[...] += jax.lax.dot_general(
            C, part.astype(jnp.bfloat16), (((1,), (0,)), ((), ())),
            preferred_element_type=jnp.float32)


def kernel(hidden_states, W_router, W_up, W_down):
    batch, seq, hidden = hidden_states.shape
    n = batch * seq
    flat = hidden_states.reshape(n, hidden)

    w1c, w2c, pos1c, pos2c, te, tot, aux = pl.pallas_call(
        _router_kernel,
        out_shape=[
            jax.ShapeDtypeStruct((n, 1), jnp.float32),
            jax.ShapeDtypeStruct((n, 1), jnp.float32),
            jax.ShapeDtypeStruct((n, 1), jnp.int32),
            jax.ShapeDtypeStruct((n, 1), jnp.int32),
            jax.ShapeDtypeStruct((RT_PAD, 1), jnp.int32),
            jax.ShapeDtypeStruct((1, 1), jnp.int32),
            jax.ShapeDtypeStruct((1, 1), jnp.float32),
        ],
    )(flat, W_router)

    pos1r = pos1c.reshape(1, n)
    pos2r = pos2c.reshape(1, n)
    w1r = w1c.reshape(1, n)
    w2r = w2c.reshape(1, n)

    sc_scatter = pl.kernel(
        _sc_scatter_body,
        out_type=jax.ShapeDtypeStruct((R, H), jnp.float32),
        mesh=plsc.VectorSubcoreMesh(core_axis_name="c", subcore_axis_name="s"),
        scratch_types=[
            pltpu.VMEM((TPW,), jnp.int32),
            pltpu.VMEM((TPW,), jnp.int32),
            pltpu.VMEM((TPW, H), jnp.float32),
            pltpu.SemaphoreType.DMA,
        ],
    )
    xs = sc_scatter(pos1c.reshape(n), pos2c.reshape(n), flat)

    te_flat = te.reshape(RT_PAD)
    tot_flat = tot.reshape(1)

    h = pl.pallas_call(
        _up_kernel,
        grid_spec=pltpu.PrefetchScalarGridSpec(
            num_scalar_prefetch=2,
            grid=(RT,),
            in_specs=[
                pl.BlockSpec((T, H), lambda r, te, tot: (r, 0)),
                pl.BlockSpec((1, INTER, H), lambda r, te, tot: (te[r], 0, 0)),
                pl.BlockSpec((1, n), lambda r, te, tot: (0, 0)),
                pl.BlockSpec((1, n), lambda r, te, tot: (0, 0)),
                pl.BlockSpec((1, n), lambda r, te, tot: (0, 0)),
                pl.BlockSpec((1, n), lambda r, te, tot: (0, 0)),
            ],
            out_specs=pl.BlockSpec((T, INTER), lambda r, te, tot: (r, 0)),
        ),
        out_shape=jax.ShapeDtypeStruct((R, INTER), jnp.bfloat16),
    )(te_flat, tot_flat, xs, W_up, pos1r, pos2r, w1r, w2r)

    out = pl.pallas_call(
        _down_combine_kernel,
        grid_spec=pltpu.PrefetchScalarGridSpec(
            num_scalar_prefetch=2,
            grid=(RT,),
            in_specs=[
                pl.BlockSpec((T, INTER), lambda r, te, tot: (r, 0)),
                pl.BlockSpec((1, H, INTER), lambda r, te, tot: (te[r], 0, 0)),
                pl.BlockSpec((n, 1), lambda r, te, tot: (0, 0)),
                pl.BlockSpec((n, 1), lambda r, te, tot: (0, 0)),
            ],
            out_specs=pl.BlockSpec((n, H), lambda r, te, tot: (0, 0)),
        ),
        out_shape=jax.ShapeDtypeStruct((n, H), jnp.float32),
    )(te_flat, tot_flat, h, W_down, pos1c, pos2c)

    return out.reshape(batch, seq, hidden), aux[0, 0]


# speedup vs baseline: 1.2641x; 1.0936x over previous
import functools

import jax
import jax.numpy as jnp
from jax import lax
from jax.experimental import pallas as pl
from jax.experimental.pallas import tpu as pltpu
from jax.experimental.pallas import tpu_sc as plsc

NE = 8
TOPK = 2
H = 1024
INTER = 4096
KI_BLK = 1024
KI = INTER // KI_BLK
T = 512
N = 2048
R = N * TOPK + NE * T
RT = R // T
RT_PAD = 32
GR = 1024


def _gelu(x):
    inner = 0.7978845608028654 * (x + 0.044715 * (x * x * x))
    return 0.5 * x * (1.0 + jnp.tanh(inner))


def _router_kernel(x_ref, wr_ref, w1_ref, w2_ref, pos1_ref, pos2_ref,
                   te_ref, tot_ref, aux_ref):
    n = x_ref.shape[0]
    logits = jax.lax.dot_general(
        x_ref[...], wr_ref[...], (((1,), (1,)), ((), ())),
        preferred_element_type=jnp.float32)
    m = jnp.max(logits, axis=-1, keepdims=True)
    ex = jnp.exp(logits - m)
    probs = ex / jnp.sum(ex, axis=-1, keepdims=True)

    lane = jax.lax.broadcasted_iota(jnp.int32, (n, NE), 1)
    m1 = jnp.max(probs, axis=-1, keepdims=True)
    a1 = jnp.min(jnp.where(probs >= m1, lane, NE), axis=-1, keepdims=True)
    oh1 = (lane == a1).astype(jnp.float32)
    p2 = jnp.where(lane == a1, -1.0, probs)
    m2 = jnp.max(p2, axis=-1, keepdims=True)
    a2 = jnp.min(jnp.where(p2 >= m2, lane, NE), axis=-1, keepdims=True)
    oh2 = (lane == a2).astype(jnp.float32)

    s = m1 + m2
    w1_ref[...] = m1 / s
    w2_ref[...] = m2 / s

    c = oh1 + oh2
    ri = jax.lax.broadcasted_iota(jnp.int32, (n, n), 0)
    ci = jax.lax.broadcasted_iota(jnp.int32, (n, n), 1)
    tri = (ci < ri).astype(jnp.float32)
    rank = jax.lax.dot_general(
        tri, c, (((1,), (0,)), ((), ())),
        preferred_element_type=jnp.float32)
    counts = jnp.sum(c, axis=0, keepdims=True)
    pc = jnp.ceil(counts / T) * T
    er = jax.lax.broadcasted_iota(jnp.int32, (NE, NE), 0)
    ec = jax.lax.broadcasted_iota(jnp.int32, (NE, NE), 1)
    off = jax.lax.dot_general(
        pc, (er < ec).astype(jnp.float32), (((1,), (0,)), ((), ())),
        preferred_element_type=jnp.float32)
    ends = jax.lax.dot_general(
        pc, (er <= ec).astype(jnp.float32), (((1,), (0,)), ((), ())),
        preferred_element_type=jnp.float32)

    base = off + rank
    pos1_ref[...] = jnp.sum(base * oh1, axis=1, keepdims=True).astype(jnp.int32)
    pos2_ref[...] = jnp.sum(base * oh2, axis=1, keepdims=True).astype(jnp.int32)

    rt = (jax.lax.broadcasted_iota(jnp.int32, (RT_PAD, NE), 0) * T).astype(
        jnp.float32)
    te = jnp.sum((rt >= ends).astype(jnp.int32), axis=1, keepdims=True)
    te_ref[...] = jnp.minimum(te, NE - 1)
    tot_ref[...] = ends[:, NE - 1:NE].astype(jnp.int32)

    f = counts / (n * TOPK)
    pmean = jnp.mean(probs, axis=0, keepdims=True)
    aux_ref[...] = jnp.full((1, 1), NE * jnp.sum(f * pmean), jnp.float32)


NWORK = 32
TPW = N // NWORK


def _sc_scatter_body(pos1_ref, pos2_ref, x_ref, xs_ref,
                     idx1_v, idx2_v, rows_v, sem):
    wid = lax.axis_index("s") * 2 + lax.axis_index("c")
    base = wid * TPW
    pltpu.sync_copy(pos1_ref.at[pl.ds(base, TPW)], idx1_v)
    pltpu.sync_copy(pos2_ref.at[pl.ds(base, TPW)], idx2_v)
    pltpu.sync_copy(x_ref.at[pl.ds(base, TPW)], rows_v)
    pltpu.async_copy(rows_v, xs_ref.at[idx1_v], sem).wait()
    pltpu.async_copy(rows_v, xs_ref.at[idx2_v], sem).wait()


def _up_kernel(te_ref, tot_ref, xs_ref, wup_ref,
               pos1_ref, pos2_ref, w1_ref, w2_ref, h_ref):
    r = pl.program_id(0)

    @pl.when(r * T < tot_ref[0])
    def _():
        wup = wup_ref[0].astype(jnp.bfloat16)
        h = jax.lax.dot_general(
            xs_ref[...].astype(jnp.bfloat16), wup, (((1,), (1,)), ((), ())),
            preferred_element_type=jnp.float32)
        ji = jax.lax.broadcasted_iota(jnp.int32, (T, N), 0) + r * T
        wrow = jnp.sum(
            jnp.where(ji == pos1_ref[...], w1_ref[...], 0.0)
            + jnp.where(ji == pos2_ref[...], w2_ref[...], 0.0),
            axis=1, keepdims=True)
        h_ref[...] = jnp.where(wrow > 0.0, _gelu(h) * wrow,
                               0.0).astype(jnp.bfloat16)


def _down_combine_kernel(te_ref, tot_ref, h_ref, wdn_ref,
                         pos1_ref, pos2_ref, out_ref):
    r = pl.program_id(0)

    @pl.when(r == 0)
    def _():
        out_ref[...] = jnp.zeros_like(out_ref)

    @pl.when(r * T < tot_ref[0])
    def _():
        wdn = wdn_ref[0].astype(jnp.bfloat16)
        part = jax.lax.dot_general(
            h_ref[...], wdn, (((1,), (1,)), ((), ())),
            preferred_element_type=jnp.float32)
        ji = jax.lax.broadcasted_iota(jnp.int32, (N, T), 1) + r * T
        C = ((ji == pos1_ref[...]) | (ji == pos2_ref[...])).astype(jnp.bfloat16)
        out_ref[...] += jax.lax.dot_general(
            C, part.astype(jnp.bfloat16), (((1,), (0,)), ((), ())),
            preferred_element_type=jnp.float32)


def kernel(hidden_states, W_router, W_up, W_down):
    batch, seq, hidden = hidden_states.shape
    n = batch * seq
    flat = hidden_states.reshape(n, hidden)

    w1c, w2c, pos1c, pos2c, te, tot, aux = pl.pallas_call(
        _router_kernel,
        out_shape=[
            jax.ShapeDtypeStruct((n, 1), jnp.float32),
            jax.ShapeDtypeStruct((n, 1), jnp.float32),
            jax.ShapeDtypeStruct((n, 1), jnp.int32),
            jax.ShapeDtypeStruct((n, 1), jnp.int32),
            jax.ShapeDtypeStruct((RT_PAD, 1), jnp.int32),
            jax.ShapeDtypeStruct((1, 1), jnp.int32),
            jax.ShapeDtypeStruct((1, 1), jnp.float32),
        ],
    )(flat, W_router)

    pos1r = pos1c.reshape(1, n)
    pos2r = pos2c.reshape(1, n)
    w1r = w1c.reshape(1, n)
    w2r = w2c.reshape(1, n)

    sc_scatter = pl.kernel(
        _sc_scatter_body,
        out_type=jax.ShapeDtypeStruct((R, H), jnp.float32),
        mesh=plsc.VectorSubcoreMesh(core_axis_name="c", subcore_axis_name="s"),
        scratch_types=[
            pltpu.VMEM((TPW,), jnp.int32),
            pltpu.VMEM((TPW,), jnp.int32),
            pltpu.VMEM((TPW, H), jnp.float32),
            pltpu.SemaphoreType.DMA,
        ],
    )
    xs = sc_scatter(pos1c.reshape(n), pos2c.reshape(n), flat)

    te_flat = te.reshape(RT_PAD)
    tot_flat = tot.reshape(1)

    h = pl.pallas_call(
        _up_kernel,
        grid_spec=pltpu.PrefetchScalarGridSpec(
            num_scalar_prefetch=2,
            grid=(RT,),
            in_specs=[
                pl.BlockSpec((T, H), lambda r, te, tot: (r, 0)),
                pl.BlockSpec((1, INTER, H), lambda r, te, tot: (te[r], 0, 0)),
                pl.BlockSpec((1, n), lambda r, te, tot: (0, 0)),
                pl.BlockSpec((1, n), lambda r, te, tot: (0, 0)),
                pl.BlockSpec((1, n), lambda r, te, tot: (0, 0)),
                pl.BlockSpec((1, n), lambda r, te, tot: (0, 0)),
            ],
            out_specs=pl.BlockSpec((T, INTER), lambda r, te, tot: (r, 0)),
        ),
        out_shape=jax.ShapeDtypeStruct((R, INTER), jnp.bfloat16),
    )(te_flat, tot_flat, xs, W_up, pos1r, pos2r, w1r, w2r)

    out = pl.pallas_call(
        _down_combine_kernel,
        grid_spec=pltpu.PrefetchScalarGridSpec(
            num_scalar_prefetch=2,
            grid=(RT,),
            in_specs=[
                pl.BlockSpec((T, INTER), lambda r, te, tot: (r, 0)),
                pl.BlockSpec((1, H, INTER), lambda r, te, tot: (te[r], 0, 0)),
                pl.BlockSpec((n, 1), lambda r, te, tot: (0, 0)),
                pl.BlockSpec((n, 1), lambda r, te, tot: (0, 0)),
            ],
            out_specs=pl.BlockSpec((n, H), lambda r, te, tot: (0, 0)),
        ),
        out_shape=jax.ShapeDtypeStruct((n, H), jnp.float32),
    )(te_flat, tot_flat, h, W_down, pos1c, pos2c)

    return out.reshape(batch, seq, hidden), aux[0, 0]
